# baseline (speedup 1.0000x reference)
import functools

import jax
import jax.numpy as jnp
from jax import lax
from jax.experimental import pallas as pl
from jax.experimental.pallas import tpu as pltpu
from jax.experimental.pallas import tpu_sc as plsc

B, T, D = 8, 2048, 1024
HID = 512
C = 256
K = max(1, min(T, int(round(T * 0.1))))
CW = float(1.0 / K) / (1.0 + 1e-8)

NGRP = T // 16
TQ = T // 4

ROWS_TC = 512


def _tc_body(x_ref, wc_ref, bc_ref, w2t_ref, b2_ref, wa2_ref, ba2_ref,
             seg_ref, sc_ref):
    hc = jnp.dot(x_ref[...], wc_ref[...],
                 preferred_element_type=jnp.float32) + bc_ref[...]
    h = jnp.maximum(hc[:, :HID], 0.0)
    ha = jnp.tanh(hc[:, HID:])
    seg_ref[...] = jnp.dot(h, w2t_ref[...],
                           preferred_element_type=jnp.float32) + b2_ref[...]
    sc_ref[...] = jnp.dot(ha, wa2_ref[...],
                          preferred_element_type=jnp.float32) + ba2_ref[0, 0]


def _tc_forward(x2, Wc, bc, W2t, b2r, wa2r, ba2r):
    grid = (x2.shape[0] // ROWS_TC,)
    return pl.pallas_call(
        _tc_body,
        grid=grid,
        in_specs=[
            pl.BlockSpec((ROWS_TC, D), lambda i: (i, 0)),
            pl.BlockSpec((D, 2 * HID), lambda i: (0, 0)),
            pl.BlockSpec((1, 2 * HID), lambda i: (0, 0)),
            pl.BlockSpec((HID, C), lambda i: (0, 0)),
            pl.BlockSpec((1, C), lambda i: (0, 0)),
            pl.BlockSpec((HID, 1), lambda i: (0, 0)),
            pl.BlockSpec((1, 1), lambda i: (0, 0)),
        ],
        out_specs=[
            pl.BlockSpec((ROWS_TC, C), lambda i: (i, 0)),
            pl.BlockSpec((ROWS_TC, 1), lambda i: (i, 0)),
        ],
        out_shape=[
            jax.ShapeDtypeStruct((x2.shape[0], C), jnp.float32),
            jax.ShapeDtypeStruct((x2.shape[0], 1), jnp.float32),
        ],
        compiler_params=pltpu.CompilerParams(
            dimension_semantics=("arbitrary",)),
    )(x2, Wc, bc, W2t, b2r, wa2r, ba2r)


def _clip_body(w_ref, seg_ref, out_ref):
    res = jnp.dot(w_ref[0], seg_ref[...],
                  preferred_element_type=jnp.float32)
    out_ref[...] = res.reshape(1, 1, C)


def _tc_clip(weights, seg2):
    out = pl.pallas_call(
        _clip_body,
        grid=(B,),
        in_specs=[
            pl.BlockSpec((1, 1, T), lambda i: (i, 0, 0)),
            pl.BlockSpec((T, C), lambda i: (i, 0)),
        ],
        out_specs=pl.BlockSpec((1, 1, C), lambda i: (i, 0, 0)),
        out_shape=jax.ShapeDtypeStruct((B, 1, C), jnp.float32),
        compiler_params=pltpu.CompilerParams(
            dimension_semantics=("arbitrary",)),
    )(weights.reshape(B, 1, T), seg2)
    return out.reshape(B, C)


def _lane_total(v):
    tot = jnp.int32(0)
    for i in range(16):
        tot = tot + lax.squeeze(lax.slice(v, (i,), (i + 1,)), (0,))
    return tot


def _sc_body(scores_hbm, w_hbm, sc_v, key_v, wrow_v):
    cid = lax.axis_index("c")
    sid = lax.axis_index("s")
    b = cid * 4 + sid // 4
    j = sid % 4

    pltpu.sync_copy(scores_hbm.at[b], sc_v)

    signbit = jnp.full((16,), 0x80000000, jnp.uint32)
    zero_u = jnp.full((16,), 0, jnp.uint32)
    one_i = jnp.full((16,), 1, jnp.int32)
    zero_i = jnp.zeros((16,), jnp.int32)
    cw_vec = jnp.full((16,), CW, jnp.float32)
    zero_f = jnp.zeros((16,), jnp.float32)

    def mk_key(g, carry):
        v = sc_v[pl.ds(g * 16, 16)]
        u = lax.bitcast_convert_type(v, jnp.uint32)
        neg = (u & signbit) != zero_u
        key_v[pl.ds(g * 16, 16)] = jnp.where(neg, ~u, u | signbit)
        return carry
    lax.fori_loop(0, NGRP, mk_key, 0)

    def count_ge(cand):
        cand_vec = jnp.full((16,), cand, dtype=jnp.uint32)
        def cbody(g, cnt):
            chunk = key_v[pl.ds(g * 16, 16)]
            return cnt + jnp.where(chunk >= cand_vec, one_i, zero_i)
        return _lane_total(lax.fori_loop(0, NGRP, cbody, zero_i, unroll=4))

    def sbody(i, carry):
        def refine(c):
            tau, cnt = c
            bit = jnp.uint32(31) - i.astype(jnp.uint32)
            cand = tau | (jnp.uint32(1) << bit)
            c2 = count_ge(cand)
            take = c2 >= K
            return (jnp.where(take, cand, tau), jnp.where(take, c2, cnt))
        return lax.cond(carry[1] != K, refine, lambda c: c, carry)

    tau, count_ge_tot = lax.fori_loop(
        0, 32, sbody, (jnp.uint32(0), jnp.int32(T)))

    tau_vec = jnp.full((16,), tau, dtype=jnp.uint32)

    @pl.when(count_ge_tot == K)
    def _():
        def wbody(g, carry):
            chunk = key_v[pl.ds(g * 16, 16)]
            sel = chunk >= tau_vec
            wrow_v[pl.ds(g * 16, 16)] = jnp.where(sel, cw_vec, zero_f)
            return carry
        lax.fori_loop(j * (NGRP // 4), (j + 1) * (NGRP // 4), wbody, 0)

    @pl.when(count_ge_tot != K)
    def _():
        def cgtbody(g, cnt):
            chunk = key_v[pl.ds(g * 16, 16)]
            return cnt + jnp.where(chunk > tau_vec, one_i, zero_i)
        count_gt = _lane_total(
            lax.fori_loop(0, NGRP, cgtbody, zero_i, unroll=4))

        lane = lax.iota(jnp.int32, 16)
        onehots = [
            jnp.where(lane == jnp.full((16,), i, jnp.int32), one_i, zero_i)
            for i in range(16)
        ]

        def wbody(g, need_rem):
            chunk = key_v[pl.ds(g * 16, 16)]
            gt = chunk > tau_vec
            eq = chunk == tau_vec
            eqc = jnp.where(eq, one_i, zero_i)
            msk = zero_i
            for i in range(16):
                e_i = lax.squeeze(lax.slice(eqc, (i,), (i + 1,)), (0,))
                t_i = jnp.where(
                    jnp.logical_and(e_i > 0, need_rem > 0),
                    jnp.int32(1), jnp.int32(0))
                need_rem = need_rem - t_i
                msk = msk + onehots[i] * jnp.full((16,), t_i, jnp.int32)
            sel = jnp.logical_or(gt, msk > zero_i)
            in_quarter = jnp.logical_and(g >= j * (NGRP // 4),
                                         g < (j + 1) * (NGRP // 4))
            @pl.when(in_quarter)
            def _():
                wrow_v[pl.ds(g * 16, 16)] = jnp.where(sel, cw_vec, zero_f)
            return need_rem

        lax.fori_loop(0, NGRP, wbody, K - count_gt)

    pltpu.sync_copy(wrow_v.at[pl.ds(j * TQ, TQ)],
                    w_hbm.at[b, pl.ds(j * TQ, TQ)])


@functools.cache
def _sc_topk():
    return pl.kernel(
        _sc_body,
        out_type=jax.ShapeDtypeStruct((B, T), jnp.float32),
        mesh=plsc.VectorSubcoreMesh(core_axis_name="c",
                                    subcore_axis_name="s"),
        scratch_types=[
            pltpu.VMEM((T,), jnp.float32),
            pltpu.VMEM((T,), jnp.uint32),
            pltpu.VMEM((T,), jnp.float32),
        ],
    )


def kernel(x, W1, b1, W2, b2, Wa1, ba1, Wa2, ba2):
    x2 = x.reshape(B * T, D)
    Wc = jnp.concatenate([W1.T, Wa1.T], axis=1)
    bc = jnp.concatenate([b1, ba1]).reshape(1, 2 * HID)
    W2t = W2.T
    b2r = b2.reshape(1, C)
    wa2r = Wa2.reshape(HID, 1)
    ba2r = ba2.reshape(1, 1)

    seg2, sc2 = _tc_forward(x2, Wc, bc, W2t, b2r, wa2r, ba2r)
    seg_logits = seg2.reshape(B, T, C)
    scores = sc2.reshape(B, T)

    weights = _sc_topk()(scores)
    clip_logits = _tc_clip(weights, seg2)
    return clip_logits, seg_logits, weights
